# baseline (speedup 1.0000x reference)
import jax
import jax.numpy as jnp
from jax import lax
from jax.experimental import pallas as pl
from jax.experimental.pallas import tpu as pltpu
from jax.experimental.pallas import tpu_sc as plsc

NC = 2
NS = 16
NW = NC * NS
L = 16
CP = 8


def _pick_block(ew: int) -> int:
    best = L
    for b in range(L, 1001, L):
        if ew % b == 0:
            best = b
    return best


def kernel(charges, cell, positions, neighbor_indices, neighbor_distances):
    n, c = charges.shape
    e = neighbor_indices.shape[0]
    assert e % NW == 0, e
    ew = e // NW
    blk = _pick_block(ew)
    nblk = ew // blk

    mesh = plsc.VectorSubcoreMesh(
        core_axis_name="c", subcore_axis_name="s", num_cores=NC, num_subcores=NS)

    def body(q_hbm, nidx_hbm, ndist_hbm, zeros_hbm, out_hbm,
             q_sp, acc_sp, idx_blk, w_blk, idxi, idxj, rows_a, rows_b):
        cid = lax.axis_index("c")
        sid = lax.axis_index("s")
        wid = cid * NS + sid

        @pl.when(sid == 0)
        def _stage():
            pltpu.sync_copy(q_hbm, q_sp)
            pltpu.sync_copy(zeros_hbm, acc_sp)

        plsc.subcore_barrier()

        iota = lax.iota(jnp.int32, L)
        pat = iota // CP
        colpat = iota - pat * CP
        col0 = iota * 0
        col1 = col0 + 1
        base = wid * ew

        def block(b, carry):
            off = base + b * blk
            pltpu.sync_copy(nidx_hbm.at[pl.ds(off, blk)], idx_blk)
            pltpu.sync_copy(ndist_hbm.at[pl.ds(off, blk)], w_blk)

            def grp(g, carry2):
                s = g * L
                e16 = s + iota
                idxi[pl.ds(s, L)] = plsc.load_gather(idx_blk, [e16, col0])
                idxj[pl.ds(s, L)] = plsc.load_gather(idx_blk, [e16, col1])
                w_blk[pl.ds(s, L)] = 0.5 / w_blk[pl.ds(s, L)]
                return carry2

            lax.fori_loop(0, blk // L, grp, 0)

            pltpu.sync_copy(q_sp.at[idxj], rows_a)
            pltpu.sync_copy(q_sp.at[idxi], rows_b)

            def sgrp(g, carry2):
                r = g * (L // CP)
                row_idx = r + pat
                w16 = plsc.load_gather(w_blk, [row_idx])
                va = plsc.load_gather(rows_a, [row_idx, colpat]) * w16
                plsc.store_scatter(rows_a, [row_idx, colpat], va)
                vb = plsc.load_gather(rows_b, [row_idx, colpat]) * w16
                plsc.store_scatter(rows_b, [row_idx, colpat], vb)
                return carry2

            lax.fori_loop(0, (blk * CP) // L, sgrp, 0)

            pltpu.sync_copy(rows_a, acc_sp.at[idxi], add=True)
            pltpu.sync_copy(rows_b, acc_sp.at[idxj], add=True)
            return carry

        lax.fori_loop(0, nblk, block, 0)

        plsc.subcore_barrier()

        @pl.when(sid == 0)
        def _writeout():
            pltpu.sync_copy(acc_sp, out_hbm.at[pl.ds(cid * n, n)])

    kfn = pl.kernel(
        body,
        out_type=jax.ShapeDtypeStruct((NC * n, CP), jnp.float32),
        mesh=mesh,
        compiler_params=pltpu.CompilerParams(
            needs_layout_passes=False, use_tc_tiling_on_sc=False),
        scratch_types=[
            pltpu.VMEM_SHARED((n, CP), jnp.float32),
            pltpu.VMEM_SHARED((n, CP), jnp.float32),
            pltpu.VMEM((blk, 2), jnp.int32),
            pltpu.VMEM((blk,), jnp.float32),
            pltpu.VMEM((blk,), jnp.int32),
            pltpu.VMEM((blk,), jnp.int32),
            pltpu.VMEM((blk, CP), jnp.float32),
            pltpu.VMEM((blk, CP), jnp.float32),
        ],
    )

    qpad = jnp.pad(charges, ((0, 0), (0, CP - c)))
    zeros = jnp.zeros((n, CP), jnp.float32)
    partial = kfn(qpad, neighbor_indices, neighbor_distances, zeros)
    return partial[:n, :c] + partial[n:, :c]


# speedup vs baseline: 5.9566x; 5.9566x over previous
import jax
import jax.numpy as jnp
from jax import lax
from jax.experimental import pallas as pl
from jax.experimental.pallas import tpu as pltpu
from jax.experimental.pallas import tpu_sc as plsc

NC = 2
NS = 16
NW = NC * NS
L = 16
CP = 8


def _pick_block(ew: int) -> int:
    best = L
    for b in range(L, 1601, L):
        if ew % b == 0:
            best = b
    return best


def kernel(charges, cell, positions, neighbor_indices, neighbor_distances):
    n, c = charges.shape
    e = neighbor_indices.shape[0]
    assert e % NW == 0, e
    ew = e // NW
    blk = _pick_block(ew)
    nblk = ew // blk

    mesh = plsc.VectorSubcoreMesh(
        core_axis_name="c", subcore_axis_name="s", num_cores=NC, num_subcores=NS)

    def body(q_hbm, ii_hbm, jj_hbm, ndist_hbm, zeros_hbm, out_hbm,
             q_sp, acc_sp, w_blk, idxi, idxj, rows_a, rows_b):
        cid = lax.axis_index("c")
        sid = lax.axis_index("s")
        wid = cid * NS + sid

        @pl.when(sid == 0)
        def _stage():
            pltpu.sync_copy(q_hbm, q_sp)
            pltpu.sync_copy(zeros_hbm, acc_sp)

        plsc.subcore_barrier()

        iota = lax.iota(jnp.int32, L)
        pat = iota // CP
        colpat = iota - pat * CP
        base = wid * ew

        def block(b, carry):
            off = base + b * blk
            pltpu.sync_copy(ii_hbm.at[pl.ds(off, blk)], idxi)
            pltpu.sync_copy(jj_hbm.at[pl.ds(off, blk)], idxj)
            pltpu.sync_copy(ndist_hbm.at[pl.ds(off, blk)], w_blk)

            def grp(g, carry2):
                s = g * L
                w_blk[pl.ds(s, L)] = 0.5 / w_blk[pl.ds(s, L)]
                return carry2

            lax.fori_loop(0, blk // L, grp, 0)

            pltpu.sync_copy(q_sp.at[idxj], rows_a)
            pltpu.sync_copy(q_sp.at[idxi], rows_b)

            def sgrp(g, carry2):
                r = g * (L // CP)
                row_idx = r + pat
                w16 = plsc.load_gather(w_blk, [row_idx])
                va = plsc.load_gather(rows_a, [row_idx, colpat]) * w16
                plsc.store_scatter(rows_a, [row_idx, colpat], va)
                vb = plsc.load_gather(rows_b, [row_idx, colpat]) * w16
                plsc.store_scatter(rows_b, [row_idx, colpat], vb)
                return carry2

            lax.fori_loop(0, (blk * CP) // L, sgrp, 0)

            pltpu.sync_copy(rows_a, acc_sp.at[idxi], add=True)
            pltpu.sync_copy(rows_b, acc_sp.at[idxj], add=True)
            return carry

        lax.fori_loop(0, nblk, block, 0)

        plsc.subcore_barrier()

        @pl.when(sid == 0)
        def _writeout():
            pltpu.sync_copy(acc_sp, out_hbm.at[pl.ds(cid * n, n)])

    kfn = pl.kernel(
        body,
        out_type=jax.ShapeDtypeStruct((NC * n, CP), jnp.float32),
        mesh=mesh,
        compiler_params=pltpu.CompilerParams(
            needs_layout_passes=False, use_tc_tiling_on_sc=False),
        scratch_types=[
            pltpu.VMEM_SHARED((n, CP), jnp.float32),
            pltpu.VMEM_SHARED((n, CP), jnp.float32),
            pltpu.VMEM((blk,), jnp.float32),
            pltpu.VMEM((blk,), jnp.int32),
            pltpu.VMEM((blk,), jnp.int32),
            pltpu.VMEM((blk, CP), jnp.float32),
            pltpu.VMEM((blk, CP), jnp.float32),
        ],
    )

    qpad = jnp.pad(charges, ((0, 0), (0, CP - c)))
    zeros = jnp.zeros((n, CP), jnp.float32)
    partial = kfn(qpad, neighbor_indices[:, 0], neighbor_indices[:, 1],
                  neighbor_distances, zeros)
    return partial[:n, :c] + partial[n:, :c]


# speedup vs baseline: 14.9541x; 2.5105x over previous
import jax
import jax.numpy as jnp
from jax import lax
from jax.experimental import pallas as pl
from jax.experimental.pallas import tpu as pltpu
from jax.experimental.pallas import tpu_sc as plsc

NC = 2
NS = 16
NW = NC * NS
L = 16
CP = 8


def _pick_block(ew: int) -> int:
    best = L
    for b in range(L, 801, L):
        if ew % b == 0 and (ew // b) % 2 == 0:
            best = b
    return best


def kernel(charges, cell, positions, neighbor_indices, neighbor_distances):
    n, c = charges.shape
    e = neighbor_indices.shape[0]
    assert e % NW == 0, e
    ew = e // NW
    blk = _pick_block(ew)
    npair = ew // blk // 2

    mesh = plsc.VectorSubcoreMesh(
        core_axis_name="c", subcore_axis_name="s", num_cores=NC, num_subcores=NS)

    def body(q_hbm, ii_hbm, jj_hbm, ndist_hbm, zeros_hbm, out_hbm,
             q_sp, acc_sp,
             w0, ia0, ja0, ra0, rb0,
             w1, ia1, ja1, ra1, rb1,
             slin0, slin1, sg0, sg1, ss0, ss1):
        cid = lax.axis_index("c")
        sid = lax.axis_index("s")
        wid = cid * NS + sid

        @pl.when(sid == 0)
        def _stage():
            pltpu.sync_copy(q_hbm, q_sp)
            pltpu.sync_copy(zeros_hbm, acc_sp)

        plsc.subcore_barrier()

        iota = lax.iota(jnp.int32, L)
        pat = iota // CP
        colpat = iota - pat * CP
        base = wid * ew

        def linload(off, wb, iab, jab, sem):
            pltpu.async_copy(ii_hbm.at[pl.ds(off, blk)], iab, sem)
            pltpu.async_copy(jj_hbm.at[pl.ds(off, blk)], jab, sem)
            pltpu.async_copy(ndist_hbm.at[pl.ds(off, blk)], wb, sem)

        def linwait(off, wb, iab, jab, sem):
            pltpu.make_async_copy(ii_hbm.at[pl.ds(off, blk)], iab, sem).wait()
            pltpu.make_async_copy(jj_hbm.at[pl.ds(off, blk)], jab, sem).wait()
            pltpu.make_async_copy(ndist_hbm.at[pl.ds(off, blk)], wb, sem).wait()

        def wscale(wb):
            @plsc.parallel_loop(0, blk // L, unroll=4)
            def _(g):
                s = g * L
                wb[pl.ds(s, L)] = 0.5 / wb[pl.ds(s, L)]

        def scale(wb, rab, rbb):
            @plsc.parallel_loop(0, (blk * CP) // L, unroll=4)
            def _(g):
                r = g * (L // CP)
                row_idx = r + pat
                w16 = plsc.load_gather(wb, [row_idx])
                va = plsc.load_gather(rab, [row_idx, colpat]) * w16
                plsc.store_scatter(rab, [row_idx, colpat], va)
                vb = plsc.load_gather(rbb, [row_idx, colpat]) * w16
                plsc.store_scatter(rbb, [row_idx, colpat], vb)

        def phase(off, wb, iab, jab, rab, rbb, slin, sg):
            linwait(off, wb, iab, jab, slin)
            pltpu.async_copy(q_sp.at[jab], rab, sg)
            pltpu.async_copy(q_sp.at[iab], rbb, sg)
            wscale(wb)
            pltpu.make_async_copy(q_sp.at[jab], rab, sg).wait()
            pltpu.make_async_copy(q_sp.at[iab], rbb, sg).wait()
            scale(wb, rab, rbb)

        def scatter(rab, rbb, iab, jab, ss):
            pltpu.async_copy(rab, acc_sp.at[iab], ss, add=True)
            pltpu.async_copy(rbb, acc_sp.at[jab], ss, add=True)

        def scatwait(rab, rbb, iab, jab, ss):
            pltpu.make_async_copy(rab, acc_sp.at[iab], ss).wait()
            pltpu.make_async_copy(rbb, acc_sp.at[jab], ss).wait()

        linload(base, w0, ia0, ja0, slin0)

        def pair(b2, carry):
            off0 = base + (2 * b2) * blk
            off1 = off0 + blk
            phase(off0, w0, ia0, ja0, ra0, rb0, slin0, sg0)

            @pl.when(b2 >= 1)
            def _():
                scatwait(ra1, rb1, ia1, ja1, ss1)

            scatter(ra0, rb0, ia0, ja0, ss0)
            linload(off1, w1, ia1, ja1, slin1)

            phase(off1, w1, ia1, ja1, ra1, rb1, slin1, sg1)
            scatter(ra1, rb1, ia1, ja1, ss1)
            scatwait(ra0, rb0, ia0, ja0, ss0)

            @pl.when(b2 + 1 < npair)
            def _():
                linload(base + (2 * b2 + 2) * blk, w0, ia0, ja0, slin0)

            return carry

        lax.fori_loop(0, npair, pair, 0)
        scatwait(ra1, rb1, ia1, ja1, ss1)

        plsc.subcore_barrier()

        @pl.when(sid == 0)
        def _writeout():
            pltpu.sync_copy(acc_sp, out_hbm.at[pl.ds(cid * n, n)])

    kfn = pl.kernel(
        body,
        out_type=jax.ShapeDtypeStruct((NC * n, CP), jnp.float32),
        mesh=mesh,
        compiler_params=pltpu.CompilerParams(
            needs_layout_passes=False, use_tc_tiling_on_sc=False),
        scratch_types=[
            pltpu.VMEM_SHARED((n, CP), jnp.float32),
            pltpu.VMEM_SHARED((n, CP), jnp.float32),
            pltpu.VMEM((blk,), jnp.float32),
            pltpu.VMEM((blk,), jnp.int32),
            pltpu.VMEM((blk,), jnp.int32),
            pltpu.VMEM((blk, CP), jnp.float32),
            pltpu.VMEM((blk, CP), jnp.float32),
            pltpu.VMEM((blk,), jnp.float32),
            pltpu.VMEM((blk,), jnp.int32),
            pltpu.VMEM((blk,), jnp.int32),
            pltpu.VMEM((blk, CP), jnp.float32),
            pltpu.VMEM((blk, CP), jnp.float32),
            pltpu.SemaphoreType.DMA,
            pltpu.SemaphoreType.DMA,
            pltpu.SemaphoreType.DMA,
            pltpu.SemaphoreType.DMA,
            pltpu.SemaphoreType.DMA,
            pltpu.SemaphoreType.DMA,
        ],
    )

    qpad = jnp.pad(charges, ((0, 0), (0, CP - c)))
    zeros = jnp.zeros((n, CP), jnp.float32)
    partial = kfn(qpad, neighbor_indices[:, 0], neighbor_indices[:, 1],
                  neighbor_distances, zeros)
    return partial[:n, :c] + partial[n:, :c]
